# Initial kernel scaffold; baseline (speedup 1.0000x reference)
import jax
import jax.numpy as jnp
from jax.experimental import pallas as pl


def kernel(input_ids, token_embedding, position_embedding):
    raise NotImplementedError("write your pallas kernel here")


# speedup vs baseline: 1.2477x; 1.2477x over previous
import functools

import jax
import jax.numpy as jnp
from jax import lax
from jax.experimental import pallas as pl
from jax.experimental.pallas import tpu as pltpu
from jax.experimental.pallas import tpu_sc as plsc

VOCAB = 49408
HIDDEN = 768
SEQ = 77
BATCH = 1024

NC = 2
NS = 16
LANES = 16
NW = NC * NS
BPW = BATCH // NW
KV = HIDDEN // LANES


def _body(ids_hbm, dest_hbm, table_hbm, pos_hbm, out_hbm,
          ids_v, dest_v, pos_v, rows0, rows1, sem0, sem1):
  wid = lax.axis_index("s") * NC + lax.axis_index("c")

  pltpu.sync_copy(ids_hbm.at[wid], ids_v)
  pltpu.sync_copy(dest_hbm.at[wid], dest_v)
  pltpu.sync_copy(pos_hbm, pos_v)

  def gather_start(s, rows, sem):
    pltpu.async_copy(table_hbm.at[ids_v.at[s]], rows, sem)

  def gather_wait(s, rows, sem):
    pltpu.make_async_copy(table_hbm.at[ids_v.at[s]], rows, sem).wait()

  def add_pos(s, rows):
    def kbody(k, _):
      pv = pos_v[pl.ds(s * HIDDEN + k * LANES, LANES)]
      def jbody(j, _):
        r = rows[j, pl.ds(k * LANES, LANES)]
        rows[j, pl.ds(k * LANES, LANES)] = r + pv
        return 0
      return lax.fori_loop(0, BPW, jbody, 0, unroll=8)
    lax.fori_loop(0, KV, kbody, 0)

  def step(s, rows, sem):
    gather_wait(s, rows, sem)
    add_pos(s, rows)
    pltpu.async_copy(rows, out_hbm.at[dest_v.at[s]], sem)
    pltpu.make_async_copy(rows, out_hbm.at[dest_v.at[s]], sem).wait()
    @pl.when(s + 2 < SEQ)
    def _():
      gather_start(s + 2, rows, sem)

  gather_start(0, rows0, sem0)
  gather_start(1, rows1, sem1)

  def loop_body(t, _):
    step(2 * t, rows0, sem0)
    step(2 * t + 1, rows1, sem1)
    return 0
  lax.fori_loop(0, (SEQ - 1) // 2, loop_body, 0)
  step(SEQ - 1, rows0, sem0)


@functools.partial(jax.jit, donate_argnums=())
def _embed(ids_w, dest_w, table, pos):
  mesh = plsc.VectorSubcoreMesh(
      core_axis_name="c", subcore_axis_name="s",
      num_cores=NC, num_subcores=NS)
  run = pl.kernel(
      _body,
      out_type=jax.ShapeDtypeStruct((BATCH * SEQ, HIDDEN), jnp.float32),
      mesh=mesh,
      scratch_types=[
          pltpu.VMEM((SEQ, BPW), jnp.int32),
          pltpu.VMEM((SEQ, BPW), jnp.int32),
          pltpu.VMEM((SEQ * HIDDEN,), jnp.float32),
          pltpu.VMEM((BPW, HIDDEN), jnp.float32),
          pltpu.VMEM((BPW, HIDDEN), jnp.float32),
          pltpu.SemaphoreType.DMA,
          pltpu.SemaphoreType.DMA,
      ],
  )
  return run(ids_w, dest_w, table, pos)


def kernel(input_ids, token_embedding, position_embedding):
  ids32 = input_ids.astype(jnp.int32)
  ids_w = ids32.reshape(NW, BPW, SEQ).transpose(0, 2, 1)
  b = jnp.arange(BATCH, dtype=jnp.int32).reshape(NW, BPW)
  dest_w = b[:, None, :] * SEQ + jnp.arange(SEQ, dtype=jnp.int32)[None, :, None]
  pos_flat = position_embedding.reshape(SEQ * HIDDEN)
  out = _embed(ids_w, dest_w, token_embedding, pos_flat)
  return out.reshape(BATCH, SEQ, HIDDEN)


# speedup vs baseline: 3.3192x; 2.6602x over previous
import functools

import jax
import jax.numpy as jnp
from jax import lax
from jax.experimental import pallas as pl
from jax.experimental.pallas import tpu as pltpu
from jax.experimental.pallas import tpu_sc as plsc

VOCAB = 49408
HIDDEN = 768
SEQ = 77
BATCH = 1024

NC = 2
NS = 16
LANES = 16
NW = NC * NS
BPW = BATCH // NW
KV = HIDDEN // LANES


def _body(ids_hbm, dest_hbm, table_hbm, pos_hbm, out_hbm,
          ids_v, dest_v, pos_v, rows0, rows1, sem0, sem1):
  wid = lax.axis_index("s") * NC + lax.axis_index("c")

  pltpu.sync_copy(ids_hbm.at[wid], ids_v)
  pltpu.sync_copy(dest_hbm.at[wid], dest_v)
  pltpu.sync_copy(pos_hbm, pos_v)

  def gather_start(s, rows, sem):
    pltpu.async_copy(table_hbm.at[ids_v.at[s]], rows, sem)

  def gather_wait(s, rows, sem):
    pltpu.make_async_copy(table_hbm.at[ids_v.at[s]], rows, sem).wait()

  def add_pos(s, rows):
    def kbody(k, _):
      pv = pos_v[pl.ds(s * HIDDEN + k * LANES, LANES)]
      def jbody(j, _):
        r = rows[j, pl.ds(k * LANES, LANES)]
        rows[j, pl.ds(k * LANES, LANES)] = r + pv
        return 0
      return lax.fori_loop(0, BPW, jbody, 0, unroll=8)
    lax.fori_loop(0, KV, kbody, 0)

  def step(s, rows, sem):
    gather_wait(s, rows, sem)
    add_pos(s, rows)
    pltpu.async_copy(rows, out_hbm.at[dest_v.at[s]], sem)
    pltpu.make_async_copy(rows, out_hbm.at[dest_v.at[s]], sem).wait()
    @pl.when(s + 2 < SEQ)
    def _():
      gather_start(s + 2, rows, sem)

  gather_start(0, rows0, sem0)
  gather_start(1, rows1, sem1)

  def loop_body(t, _):
    step(2 * t, rows0, sem0)
    step(2 * t + 1, rows1, sem1)
    return 0
  lax.fori_loop(0, (SEQ - 1) // 2, loop_body, 0)
  step(SEQ - 1, rows0, sem0)


@functools.partial(jax.jit, donate_argnums=())
def _embed(ids_w, dest_w, table, pos):
  mesh = plsc.VectorSubcoreMesh(
      core_axis_name="c", subcore_axis_name="s",
      num_cores=NC, num_subcores=NS)
  run = pl.kernel(
      _body,
      out_type=jax.ShapeDtypeStruct((BATCH * SEQ, HIDDEN), jnp.float32),
      mesh=mesh,
      scratch_types=[
          pltpu.VMEM((SEQ, BPW), jnp.int32),
          pltpu.VMEM((SEQ, BPW), jnp.int32),
          pltpu.VMEM((SEQ * HIDDEN,), jnp.float32),
          pltpu.VMEM((BPW, HIDDEN), jnp.float32),
          pltpu.VMEM((BPW, HIDDEN), jnp.float32),
          pltpu.SemaphoreType.DMA,
          pltpu.SemaphoreType.DMA,
      ],
  )
  return run(ids_w, dest_w, table, pos)


def kernel(input_ids, token_embedding, position_embedding):
  ids32 = input_ids.astype(jnp.int32)
  ids_w = ids32.reshape(NW, BPW, SEQ).transpose(0, 2, 1)
  b = jnp.arange(BATCH, dtype=jnp.int32).reshape(NW, BPW)
  dest_w = b[:, None, :] + (jnp.arange(SEQ, dtype=jnp.int32) * BATCH)[None, :, None]
  pos_flat = position_embedding.reshape(SEQ * HIDDEN)
  out = _embed(ids_w, dest_w, token_embedding, pos_flat)
  return out.reshape(SEQ, BATCH, HIDDEN).transpose(1, 0, 2)


# speedup vs baseline: 3.3569x; 1.0113x over previous
import functools

import jax
import jax.numpy as jnp
from jax import lax
from jax.experimental import pallas as pl
from jax.experimental.pallas import tpu as pltpu
from jax.experimental.pallas import tpu_sc as plsc

VOCAB = 49408
HIDDEN = 768
SEQ = 77
BATCH = 1024

NC = 2
NS = 16
LANES = 16
NW = NC * NS
BPW = BATCH // NW
KV = HIDDEN // LANES


def _body(ids_hbm, table_hbm, pos_hbm, out_hbm,
          ids_v, pos_v, rows0, rows1, sem0, sem1):
  wid = lax.axis_index("s") * NC + lax.axis_index("c")

  pltpu.sync_copy(ids_hbm.at[wid], ids_v)
  pltpu.sync_copy(pos_hbm, pos_v)

  def gather_start(s, rows, sem):
    pltpu.async_copy(table_hbm.at[ids_v.at[s]], rows, sem)

  def gather_wait(s, rows, sem):
    pltpu.make_async_copy(table_hbm.at[ids_v.at[s]], rows, sem).wait()

  def add_pos(s, rows):
    def kbody(k, _):
      pv = pos_v[pl.ds(s * HIDDEN + k * LANES, LANES)]
      def jbody(j, _):
        r = rows[j, pl.ds(k * LANES, LANES)]
        rows[j, pl.ds(k * LANES, LANES)] = r + pv
        return 0
      return lax.fori_loop(0, BPW, jbody, 0, unroll=8)
    lax.fori_loop(0, KV, kbody, 0)

  def step(s, rows, sem):
    gather_wait(s, rows, sem)
    add_pos(s, rows)
    dst = out_hbm.at[pl.ds(s * BATCH + wid * BPW, BPW)]
    pltpu.async_copy(rows, dst, sem)
    pltpu.make_async_copy(rows, dst, sem).wait()
    @pl.when(s + 2 < SEQ)
    def _():
      gather_start(s + 2, rows, sem)

  gather_start(0, rows0, sem0)
  gather_start(1, rows1, sem1)

  def loop_body(t, _):
    step(2 * t, rows0, sem0)
    step(2 * t + 1, rows1, sem1)
    return 0
  lax.fori_loop(0, (SEQ - 1) // 2, loop_body, 0)
  step(SEQ - 1, rows0, sem0)


@functools.partial(jax.jit, donate_argnums=())
def _embed(ids_w, table, pos):
  mesh = plsc.VectorSubcoreMesh(
      core_axis_name="c", subcore_axis_name="s",
      num_cores=NC, num_subcores=NS)
  run = pl.kernel(
      _body,
      out_type=jax.ShapeDtypeStruct((BATCH * SEQ, HIDDEN), jnp.float32),
      mesh=mesh,
      scratch_types=[
          pltpu.VMEM((SEQ, BPW), jnp.int32),
          pltpu.VMEM((SEQ * HIDDEN,), jnp.float32),
          pltpu.VMEM((BPW, HIDDEN), jnp.float32),
          pltpu.VMEM((BPW, HIDDEN), jnp.float32),
          pltpu.SemaphoreType.DMA,
          pltpu.SemaphoreType.DMA,
      ],
  )
  return run(ids_w, table, pos)


def kernel(input_ids, token_embedding, position_embedding):
  ids32 = input_ids.astype(jnp.int32)
  ids_w = ids32.reshape(NW, BPW, SEQ).transpose(0, 2, 1)
  pos_flat = position_embedding.reshape(SEQ * HIDDEN)
  out = _embed(ids_w, token_embedding, pos_flat)
  return out.reshape(SEQ, BATCH, HIDDEN).transpose(1, 0, 2)
